# Initial kernel scaffold; baseline (speedup 1.0000x reference)
import jax
import jax.numpy as jnp
from jax.experimental import pallas as pl


def kernel(x, time_embedding, gate_W, gate_b, W1, b1, W2, b2):
    raise NotImplementedError("write your pallas kernel here")


# speedup vs baseline: 4.5874x; 4.5874x over previous
import jax
import jax.numpy as jnp
import numpy as np
from jax import lax
from jax.experimental import pallas as pl
from jax.experimental.pallas import tpu as pltpu

_B, _F, _S, _P, _E, _FF, _K = 64, 7, 512, 96, 8, 2048, 2
_N = _B * _F
_ALPHA = 10.0


def _moe_body(x_ref, ti_ref, gw_ref, gb_ref, w1_ref, b1_ref, w2_ref, b2_ref,
              out_ref, loss_ref, gates_ref):
    e = pl.program_id(0)

    @pl.when(e == 0)
    def _gating_and_loss():
        logits = jnp.dot(ti_ref[...], gw_ref[...],
                         preferred_element_type=jnp.float32) + gb_ref[...]
        m1 = jnp.max(logits, axis=1, keepdims=True)
        idx = lax.broadcasted_iota(jnp.int32, (_N, _E), 1)
        first_idx = jnp.min(jnp.where(logits == m1, idx, _E), axis=1,
                            keepdims=True)
        m2 = jnp.max(jnp.where(idx == first_idx, -jnp.inf, logits), axis=1,
                     keepdims=True)
        below_topk = logits < m2
        ex = jnp.exp(logits - m1)
        sm = ex / jnp.sum(ex, axis=1, keepdims=True)
        outv = jnp.where(below_topk, _ALPHA * jnp.log(sm + 1.0),
                         _ALPHA * (jnp.exp(sm) - 1.0))
        mo = jnp.max(outv, axis=1, keepdims=True)
        exo = jnp.exp(outv - mo)
        gates = exo / jnp.sum(exo, axis=1, keepdims=True)
        gates_ref[...] = gates

        row = lax.broadcasted_iota(jnp.int32, (_F, _N), 0)
        col = lax.broadcasted_iota(jnp.int32, (_F, _N), 1)
        sel = (col % _F == row).astype(jnp.float32)
        imp = jnp.dot(sel, gates, preferred_element_type=jnp.float32)
        mean = jnp.mean(imp, axis=1, keepdims=True)
        var = jnp.sum((imp - mean) ** 2, axis=1, keepdims=True) / (_E - 1)
        loss_ref[...] = jnp.sum(var / (mean ** 2 + 1e-10),
                                keepdims=True).reshape(1, 1)

    h = jnp.dot(x_ref[...], w1_ref[0], preferred_element_type=jnp.float32)
    h = h + b1_ref[0]
    h = 0.5 * h * (1.0 + lax.erf(h * np.float32(1.0 / np.sqrt(2.0))))
    o = jnp.dot(h, w2_ref[0], preferred_element_type=jnp.float32) + b2_ref[0]
    lane = lax.broadcasted_iota(jnp.int32, (_N, _E), 1)
    g = jnp.sum(jnp.where(lane == e, gates_ref[...], 0.0), axis=1,
                keepdims=True)
    contrib = g * o

    @pl.when(e == 0)
    def _init():
        out_ref[...] = contrib

    @pl.when(e > 0)
    def _acc():
        out_ref[...] += contrib


def kernel(x, time_embedding, gate_W, gate_b, W1, b1, W2, b2):
    x_flat = x.reshape(_N, _S)
    ti_flat = time_embedding.reshape(_N, _S)
    gb = gate_b.reshape(1, _E)
    b1r = b1.reshape(_E, 1, _FF)
    b2r = b2.reshape(_E, 1, _P)

    out, loss = pl.pallas_call(
        _moe_body,
        grid=(_E,),
        in_specs=[
            pl.BlockSpec((_N, _S), lambda e: (0, 0)),
            pl.BlockSpec((_N, _S), lambda e: (0, 0)),
            pl.BlockSpec((_S, _E), lambda e: (0, 0)),
            pl.BlockSpec((1, _E), lambda e: (0, 0)),
            pl.BlockSpec((1, _S, _FF), lambda e: (e, 0, 0)),
            pl.BlockSpec((1, 1, _FF), lambda e: (e, 0, 0)),
            pl.BlockSpec((1, _FF, _P), lambda e: (e, 0, 0)),
            pl.BlockSpec((1, 1, _P), lambda e: (e, 0, 0)),
        ],
        out_specs=[
            pl.BlockSpec((_N, _P), lambda e: (0, 0)),
            pl.BlockSpec((1, 1), lambda e: (0, 0)),
        ],
        out_shape=[
            jax.ShapeDtypeStruct((_N, _P), jnp.float32),
            jax.ShapeDtypeStruct((1, 1), jnp.float32),
        ],
        scratch_shapes=[pltpu.VMEM((_N, _E), jnp.float32)],
        compiler_params=pltpu.CompilerParams(
            dimension_semantics=("arbitrary",)),
    )(x_flat, ti_flat, gate_W, gb, W1, b1r, W2, b2r)

    return out.reshape(_B, _F, _P), loss[0, 0]
